# Initial kernel scaffold; baseline (speedup 1.0000x reference)
import jax
import jax.numpy as jnp
from jax.experimental import pallas as pl


def kernel(neighbor_feature, weight):
    raise NotImplementedError("write your pallas kernel here")


# speedup vs baseline: 1.1482x; 1.1482x over previous
import functools

import jax
import jax.numpy as jnp
from jax.experimental import pallas as pl
from jax.experimental.pallas import tpu as pltpu

N = 10000
K = 32
D = 128
BLOCK = 400


def _body(x_ref, w_ref, o_ref):
    x = x_ref[...]
    s = jnp.sum(x, axis=1) * (1.0 / K)
    o_ref[...] = jnp.dot(s, w_ref[...], preferred_element_type=jnp.float32)


@jax.jit
def kernel(neighbor_feature, weight):
    grid = N // BLOCK
    return pl.pallas_call(
        _body,
        grid=(grid,),
        in_specs=[
            pl.BlockSpec((BLOCK, K, D), lambda i: (i, 0, 0)),
            pl.BlockSpec((D, D), lambda i: (0, 0)),
        ],
        out_specs=pl.BlockSpec((BLOCK, D), lambda i: (i, 0)),
        out_shape=jax.ShapeDtypeStruct((N, D), jnp.float32),
        compiler_params=pltpu.CompilerParams(
            dimension_semantics=("arbitrary",),
        ),
    )(neighbor_feature, weight)
